# baseline (speedup 1.0000x reference)
import functools
import jax
import jax.numpy as jnp
from jax.experimental import pallas as pl

N = 50000
H = 64
F = 64
ALPHA = 0.9

BLK = 1000


def _mm_relu_body(x_ref, w_ref, b_ref, o_ref):
    o_ref[...] = jax.nn.relu(
        jnp.dot(x_ref[...], w_ref[...].T, preferred_element_type=jnp.float32)
        + b_ref[...]
    )


def _mm_relu(x, W, b):
    return pl.pallas_call(
        _mm_relu_body,
        grid=(N // BLK,),
        in_specs=[
            pl.BlockSpec((BLK, H), lambda i: (i, 0)),
            pl.BlockSpec((F, H), lambda i: (0, 0)),
            pl.BlockSpec((F,), lambda i: (0,)),
        ],
        out_specs=pl.BlockSpec((BLK, F), lambda i: (i, 0)),
        out_shape=jax.ShapeDtypeStruct((N, F), jnp.float32),
    )(x, W, b)


def _layer_body(agg_ref, x0_ref, out_ref, w_ref, o_ref):
    h = (1.0 - ALPHA) * agg_ref[...] + ALPHA * x0_ref[...]
    hw = jnp.dot(h, w_ref[...], preferred_element_type=jnp.float32)
    o_ref[...] = out_ref[...] + jax.nn.relu(hw)


def _layer_dense(agg, x0, out, W):
    return pl.pallas_call(
        _layer_body,
        grid=(N // BLK,),
        in_specs=[
            pl.BlockSpec((BLK, F), lambda i: (i, 0)),
            pl.BlockSpec((BLK, F), lambda i: (i, 0)),
            pl.BlockSpec((BLK, F), lambda i: (i, 0)),
            pl.BlockSpec((F, F), lambda i: (0, 0)),
        ],
        out_specs=pl.BlockSpec((BLK, F), lambda i: (i, 0)),
        out_shape=jax.ShapeDtypeStruct((N, F), jnp.float32),
    )(agg, x0, out, W)


def kernel(x, edge_index, edge_weight, edge_attr, W0, b0, W1_0, W1_1):
    row, col = edge_index[0], edge_index[1]
    x1 = _mm_relu(x, W0, b0)
    deg = jnp.zeros((N,), dtype=jnp.float32).at[col].add(edge_weight)
    dis = jnp.where(deg > 0, jax.lax.rsqrt(jnp.where(deg > 0, deg, 1.0)), 0.0)
    norm = dis[row] * edge_weight * dis[col]
    out = x1
    for W in (W1_0, W1_1):
        msg = out[row] * norm[:, None]
        agg = jnp.zeros_like(out).at[col].add(msg)
        out = _layer_dense(agg, x1, out, W)
    return out


# speedup vs baseline: 11.3841x; 11.3841x over previous
import functools
import jax
import jax.numpy as jnp
from jax import lax
from jax.experimental import pallas as pl
from jax.experimental.pallas import tpu as pltpu
from jax.experimental.pallas import tpu_sc as plsc

N = 50000
E = 800000
H = 64
F = 64
HF = 32
ALPHA = 0.9

NC = 2
NS = 16
L = 16

CH = 400
EPT = E // NS
NPT8 = 3128
NPTL = N - (NS - 1) * NPT8

BLK = 1000

_MESH = plsc.VectorSubcoreMesh(core_axis_name="c", subcore_axis_name="s")


def _init_acc(zeros_hbm, acc_sh, s):
    off = s * NPT8

    @pl.when(s < NS - 1)
    def _():
        pltpu.sync_copy(zeros_hbm, acc_sh.at[pl.ds(off, NPT8)])

    @pl.when(s == NS - 1)
    def _():
        pltpu.sync_copy(zeros_hbm.at[pl.ds(0, NPTL)], acc_sh.at[pl.ds(off, NPTL)])


def _writeback_acc(acc_sh, out_hbm, c, s):
    off = s * NPT8

    @pl.when(s < NS - 1)
    def _():
        pltpu.sync_copy(acc_sh.at[pl.ds(off, NPT8)],
                        out_hbm.at[pl.ds(c * N + off, NPT8)])

    @pl.when(s == NS - 1)
    def _():
        pltpu.sync_copy(acc_sh.at[pl.ds(off, NPTL)],
                        out_hbm.at[pl.ds(c * N + off, NPTL)])


@functools.partial(
    pl.kernel,
    out_type=jax.ShapeDtypeStruct((NC * N, HF), jnp.float32),
    mesh=_MESH,
    compiler_params=pltpu.CompilerParams(use_tc_tiling_on_sc=False),
    scratch_types=[
        pltpu.VMEM((CH,), jnp.int32),
        pltpu.VMEM((CH,), jnp.float32),
        pltpu.VMEM((CH, HF), jnp.float32),
        pltpu.VMEM_SHARED((N, HF), jnp.float32),
    ],
)
def _deg_kernel(col_hbm, w_hbm, zeros_hbm, deg_hbm, col_v, w_v, rows_v, acc_sh):
    c = lax.axis_index("c")
    s = lax.axis_index("s")
    _init_acc(zeros_hbm, acc_sh, s)
    plsc.subcore_barrier()

    def chunk(k, carry):
        base = s * EPT + k * CH
        pltpu.sync_copy(col_hbm.at[pl.ds(base, CH)], col_v)
        pltpu.sync_copy(w_hbm.at[pl.ds(base, CH)], w_v)

        def splat(j, cc):
            wv16 = w_v[pl.ds(j * L, L)]
            for l in range(L):
                sp = jnp.full((L,), wv16[l], jnp.float32)
                i = j * L + l
                rows_v[i, pl.ds(0, L)] = sp
                rows_v[i, pl.ds(L, L)] = sp
            return cc

        lax.fori_loop(0, CH // L, splat, 0)
        pltpu.sync_copy(rows_v, acc_sh.at[col_v], add=True)
        return carry

    lax.fori_loop(0, EPT // CH, chunk, 0)
    plsc.subcore_barrier()
    _writeback_acc(acc_sh, deg_hbm, c, s)


@functools.partial(
    pl.kernel,
    out_type=jax.ShapeDtypeStruct((NC * N, HF), jnp.float32),
    mesh=_MESH,
    compiler_params=pltpu.CompilerParams(use_tc_tiling_on_sc=False),
    scratch_types=[
        pltpu.VMEM((CH,), jnp.int32),
        pltpu.VMEM((CH,), jnp.int32),
        pltpu.VMEM((CH,), jnp.float32),
        pltpu.VMEM((CH, HF), jnp.float32),
        pltpu.VMEM_SHARED((N, HF), jnp.float32),
        pltpu.SemaphoreType.DMA,
    ],
)
def _edge_kernel(row_hbm, col_hbm, w_hbm, outp2_hbm, zeros_hbm, agg_hbm,
                 gidx_v, col_v, w_v, rows_v, acc_sh, sem):
    c = lax.axis_index("c")
    s = lax.axis_index("s")
    _init_acc(zeros_hbm, acc_sh, s)
    plsc.subcore_barrier()

    cN = c * N

    def chunk(k, carry):
        base = s * EPT + k * CH
        pltpu.sync_copy(row_hbm.at[pl.ds(base, CH)], gidx_v)
        pltpu.sync_copy(col_hbm.at[pl.ds(base, CH)], col_v)
        pltpu.sync_copy(w_hbm.at[pl.ds(base, CH)], w_v)

        def addc(i, cc):
            gidx_v[pl.ds(i * L, L)] = gidx_v[pl.ds(i * L, L)] + cN
            return cc

        lax.fori_loop(0, CH // L, addc, 0)
        pltpu.async_copy(outp2_hbm.at[gidx_v], rows_v, sem).wait()

        def scale(j, cc):
            wv16 = w_v[pl.ds(j * L, L)]
            for l in range(L):
                wv = wv16[l]
                i = j * L + l
                rows_v[i, pl.ds(0, L)] = rows_v[i, pl.ds(0, L)] * wv
                rows_v[i, pl.ds(L, L)] = rows_v[i, pl.ds(L, L)] * wv
            return cc

        lax.fori_loop(0, CH // L, scale, 0)
        pltpu.sync_copy(rows_v, acc_sh.at[col_v], add=True)
        return carry

    lax.fori_loop(0, EPT // CH, chunk, 0)
    plsc.subcore_barrier()
    _writeback_acc(acc_sh, agg_hbm, c, s)


def _tcb_body(x_ref, w0_ref, b0_ref, deg_ref,
              x1_ref, dis_ref, outp2_ref):
    x1 = jax.nn.relu(
        jnp.dot(x_ref[...], w0_ref[...].T, preferred_element_type=jnp.float32)
        + b0_ref[...]
    )
    deg = deg_ref[...]
    dis = jnp.where(deg > 0, lax.rsqrt(jnp.where(deg > 0, deg, 1.0)), 0.0)
    x1_ref[...] = x1
    dis_ref[...] = dis
    outp2_ref[...] = jnp.stack([x1[:, :HF] * dis, x1[:, HF:] * dis], axis=0)


def _tcb(x, W0, b0, deg2):
    return pl.pallas_call(
        _tcb_body,
        grid=(N // BLK,),
        in_specs=[
            pl.BlockSpec((BLK, H), lambda i: (i, 0)),
            pl.BlockSpec((F, H), lambda i: (0, 0)),
            pl.BlockSpec((F,), lambda i: (0,)),
            pl.BlockSpec((BLK, HF), lambda i: (i, 0)),
        ],
        out_specs=[
            pl.BlockSpec((BLK, F), lambda i: (i, 0)),
            pl.BlockSpec((BLK, HF), lambda i: (i, 0)),
            pl.BlockSpec((NC, BLK, HF), lambda i: (0, i, 0)),
        ],
        out_shape=[
            jax.ShapeDtypeStruct((N, F), jnp.float32),
            jax.ShapeDtypeStruct((N, HF), jnp.float32),
            jax.ShapeDtypeStruct((NC, N, HF), jnp.float32),
        ],
    )(x, W0, b0, deg2)


def _tcc_body(aggA_ref, aggB_ref, dis_ref, x1_ref, out_ref, w_ref,
              o_ref, outp2_ref):
    dis = dis_ref[...]
    agg = jnp.concatenate([aggA_ref[...] * dis, aggB_ref[...] * dis], axis=1)
    h = (1.0 - ALPHA) * agg + ALPHA * x1_ref[...]
    hw = jnp.dot(h, w_ref[...], preferred_element_type=jnp.float32)
    o = out_ref[...] + jax.nn.relu(hw)
    o_ref[...] = o
    outp2_ref[...] = jnp.stack([o[:, :HF] * dis, o[:, HF:] * dis], axis=0)


def _tcc(agg2, dis, x1, out, W):
    return pl.pallas_call(
        _tcc_body,
        grid=(N // BLK,),
        in_specs=[
            pl.BlockSpec((BLK, HF), lambda i: (i, 0)),
            pl.BlockSpec((BLK, HF), lambda i: (N // BLK + i, 0)),
            pl.BlockSpec((BLK, HF), lambda i: (i, 0)),
            pl.BlockSpec((BLK, F), lambda i: (i, 0)),
            pl.BlockSpec((BLK, F), lambda i: (i, 0)),
            pl.BlockSpec((F, F), lambda i: (0, 0)),
        ],
        out_specs=[
            pl.BlockSpec((BLK, F), lambda i: (i, 0)),
            pl.BlockSpec((NC, BLK, HF), lambda i: (0, i, 0)),
        ],
        out_shape=[
            jax.ShapeDtypeStruct((N, F), jnp.float32),
            jax.ShapeDtypeStruct((NC, N, HF), jnp.float32),
        ],
    )(agg2, agg2, dis, x1, out, W)


def kernel(x, edge_index, edge_weight, edge_attr, W0, b0, W1_0, W1_1):
    row = edge_index[0]
    col = edge_index[1]
    zeros = jnp.zeros((NPT8, HF), jnp.float32)

    deg2 = _deg_kernel(col, edge_weight, zeros)
    x1, dis, outp2 = _tcb(x, W0, b0, deg2)

    out = x1
    for W in (W1_0, W1_1):
        agg2 = _edge_kernel(row, col, edge_weight,
                            outp2.reshape(NC * N, HF), zeros)
        out, outp2 = _tcc(agg2, dis, x1, out, W)
    return out


# speedup vs baseline: 15.1451x; 1.3304x over previous
import functools
import jax
import jax.numpy as jnp
from jax import lax
from jax.experimental import pallas as pl
from jax.experimental.pallas import tpu as pltpu
from jax.experimental.pallas import tpu_sc as plsc

N = 50000
E = 800000
H = 64
F = 64
HF = 32
DW = 16
ALPHA = 0.9

NC = 2
NS = 16
L = 16

CH = 400
EPT = E // NS
NCHUNK = EPT // CH

EPW = E // (NC * NS)
DEG_FULL = EPW // CH
DEG_REM_BASE = (NC * NS) * (DEG_FULL * CH)
NPT8 = 3128
NPTL = N - (NS - 1) * NPT8

BLK = 1000

_MESH = plsc.VectorSubcoreMesh(core_axis_name="c", subcore_axis_name="s")
_SC_PARAMS = pltpu.CompilerParams(use_tc_tiling_on_sc=False)


def _init_acc(zeros_hbm, acc_sh, s):
    off = s * NPT8

    @pl.when(s < NS - 1)
    def _():
        pltpu.sync_copy(zeros_hbm, acc_sh.at[pl.ds(off, NPT8)])

    @pl.when(s == NS - 1)
    def _():
        pltpu.sync_copy(zeros_hbm.at[pl.ds(0, NPTL)], acc_sh.at[pl.ds(off, NPTL)])


def _writeback_acc(acc_sh, out_hbm, c, s):
    off = s * NPT8

    @pl.when(s < NS - 1)
    def _():
        pltpu.sync_copy(acc_sh.at[pl.ds(off, NPT8)],
                        out_hbm.at[pl.ds(c * N + off, NPT8)])

    @pl.when(s == NS - 1)
    def _():
        pltpu.sync_copy(acc_sh.at[pl.ds(off, NPTL)],
                        out_hbm.at[pl.ds(c * N + off, NPTL)])


@functools.partial(
    pl.kernel,
    out_type=jax.ShapeDtypeStruct((NC * N, DW), jnp.float32),
    mesh=_MESH,
    compiler_params=_SC_PARAMS,
    scratch_types=[
        pltpu.VMEM((CH,), jnp.int32),
        pltpu.VMEM((CH,), jnp.float32),
        pltpu.VMEM((CH, DW), jnp.float32),
        pltpu.VMEM_SHARED((N, DW), jnp.float32),
    ],
)
def _deg_kernel(col_hbm, w_hbm, zeros_hbm, deg_hbm, col_v, w_v, rows_v, acc_sh):
    c = lax.axis_index("c")
    s = lax.axis_index("s")
    _init_acc(zeros_hbm, acc_sh, s)
    plsc.subcore_barrier()

    wid = c * NS + s

    def do_chunk(base):
        pltpu.sync_copy(col_hbm.at[pl.ds(base, CH)], col_v)
        pltpu.sync_copy(w_hbm.at[pl.ds(base, CH)], w_v)

        def splat(j, cc):
            wv16 = w_v[pl.ds(j * L, L)]
            for l in range(L):
                i = j * L + l
                rows_v[i, pl.ds(0, DW)] = jnp.full((DW,), wv16[l], jnp.float32)
            return cc

        lax.fori_loop(0, CH // L, splat, 0)
        pltpu.sync_copy(rows_v, acc_sh.at[col_v], add=True)

    def chunk(k, carry):
        do_chunk(wid * (DEG_FULL * CH) + k * CH)
        return carry

    lax.fori_loop(0, DEG_FULL, chunk, 0)

    @pl.when(wid < (E - DEG_REM_BASE) // CH)
    def _():
        do_chunk(DEG_REM_BASE + wid * CH)

    plsc.subcore_barrier()
    _writeback_acc(acc_sh, deg_hbm, c, s)


@functools.partial(
    pl.kernel,
    out_type=jax.ShapeDtypeStruct((NC * N, HF), jnp.float32),
    mesh=_MESH,
    compiler_params=_SC_PARAMS,
    scratch_types=[
        pltpu.VMEM((CH,), jnp.int32),
        pltpu.VMEM((CH,), jnp.int32),
        pltpu.VMEM((CH,), jnp.int32),
        pltpu.VMEM((CH,), jnp.int32),
        pltpu.VMEM((CH,), jnp.float32),
        pltpu.VMEM((CH,), jnp.float32),
        pltpu.VMEM((CH, HF), jnp.float32),
        pltpu.VMEM((CH, HF), jnp.float32),
        pltpu.VMEM_SHARED((N, HF), jnp.float32),
        pltpu.SemaphoreType.DMA,
        pltpu.SemaphoreType.DMA,
    ],
)
def _edge_kernel(row_hbm, col_hbm, w_hbm, outp2_hbm, zeros_hbm, agg_hbm,
                 gidx0, gidx1, col0, col1, w0, w1, rows0, rows1,
                 acc_sh, sem0, sem1):
    c = lax.axis_index("c")
    s = lax.axis_index("s")
    _init_acc(zeros_hbm, acc_sh, s)
    plsc.subcore_barrier()

    cN = c * N
    gidx = (gidx0, gidx1)
    colb = (col0, col1)
    wb = (w0, w1)
    rows = (rows0, rows1)
    sems = (sem0, sem1)

    def load_and_fire(k, b):
        base = s * EPT + k * CH
        pltpu.sync_copy(row_hbm.at[pl.ds(base, CH)], gidx[b])
        pltpu.sync_copy(col_hbm.at[pl.ds(base, CH)], colb[b])
        pltpu.sync_copy(w_hbm.at[pl.ds(base, CH)], wb[b])

        def addc(i, cc):
            gidx[b][pl.ds(i * L, L)] = gidx[b][pl.ds(i * L, L)] + cN
            return cc

        lax.fori_loop(0, CH // L, addc, 0, unroll=4)
        pltpu.async_copy(outp2_hbm.at[gidx[b]], rows[b], sems[b])

    def process(b):
        pltpu.make_async_copy(outp2_hbm.at[gidx[b]], rows[b], sems[b]).wait()

        def scale(j, cc):
            wv16 = wb[b][pl.ds(j * L, L)]
            for l in range(L):
                wv = wv16[l]
                i = j * L + l
                rows[b][i, pl.ds(0, L)] = rows[b][i, pl.ds(0, L)] * wv
                rows[b][i, pl.ds(L, L)] = rows[b][i, pl.ds(L, L)] * wv
            return cc

        lax.fori_loop(0, CH // L, scale, 0)
        pltpu.sync_copy(rows[b], acc_sh.at[colb[b]], add=True)

    load_and_fire(0, 0)

    def pair(k2, carry):
        k = 2 * k2
        load_and_fire(k + 1, 1)
        process(0)
        load_and_fire(k + 2, 0)
        process(1)
        return carry

    lax.fori_loop(0, (NCHUNK - 1) // 2, pair, 0)
    process(0)

    plsc.subcore_barrier()
    _writeback_acc(acc_sh, agg_hbm, c, s)


def _tcb_body(x_ref, w0_ref, b0_ref, degA_ref, degB_ref,
              x1_ref, dis_ref, outp2_ref):
    x1 = jax.nn.relu(
        jnp.dot(x_ref[...], w0_ref[...].T, preferred_element_type=jnp.float32)
        + b0_ref[...]
    )
    deg = degA_ref[...] + degB_ref[...]
    dis16 = jnp.where(deg > 0, lax.rsqrt(jnp.where(deg > 0, deg, 1.0)), 0.0)
    dis = jnp.concatenate([dis16, dis16], axis=1)
    x1_ref[...] = x1
    dis_ref[...] = dis
    outp2_ref[...] = jnp.stack([x1[:, :HF] * dis, x1[:, HF:] * dis], axis=0)


def _tcb(x, W0, b0, deg2):
    return pl.pallas_call(
        _tcb_body,
        grid=(N // BLK,),
        in_specs=[
            pl.BlockSpec((BLK, H), lambda i: (i, 0)),
            pl.BlockSpec((F, H), lambda i: (0, 0)),
            pl.BlockSpec((F,), lambda i: (0,)),
            pl.BlockSpec((BLK, DW), lambda i: (i, 0)),
            pl.BlockSpec((BLK, DW), lambda i: (N // BLK + i, 0)),
        ],
        out_specs=[
            pl.BlockSpec((BLK, F), lambda i: (i, 0)),
            pl.BlockSpec((BLK, HF), lambda i: (i, 0)),
            pl.BlockSpec((NC, BLK, HF), lambda i: (0, i, 0)),
        ],
        out_shape=[
            jax.ShapeDtypeStruct((N, F), jnp.float32),
            jax.ShapeDtypeStruct((N, HF), jnp.float32),
            jax.ShapeDtypeStruct((NC, N, HF), jnp.float32),
        ],
    )(x, W0, b0, deg2, deg2)


def _tcc_body(aggA_ref, aggB_ref, dis_ref, x1_ref, out_ref, w_ref,
              o_ref, outp2_ref):
    dis = dis_ref[...]
    agg = jnp.concatenate([aggA_ref[...] * dis, aggB_ref[...] * dis], axis=1)
    h = (1.0 - ALPHA) * agg + ALPHA * x1_ref[...]
    hw = jnp.dot(h, w_ref[...], preferred_element_type=jnp.float32)
    o = out_ref[...] + jax.nn.relu(hw)
    o_ref[...] = o
    outp2_ref[...] = jnp.stack([o[:, :HF] * dis, o[:, HF:] * dis], axis=0)


def _tcc(agg2, dis, x1, out, W):
    return pl.pallas_call(
        _tcc_body,
        grid=(N // BLK,),
        in_specs=[
            pl.BlockSpec((BLK, HF), lambda i: (i, 0)),
            pl.BlockSpec((BLK, HF), lambda i: (N // BLK + i, 0)),
            pl.BlockSpec((BLK, HF), lambda i: (i, 0)),
            pl.BlockSpec((BLK, F), lambda i: (i, 0)),
            pl.BlockSpec((BLK, F), lambda i: (i, 0)),
            pl.BlockSpec((F, F), lambda i: (0, 0)),
        ],
        out_specs=[
            pl.BlockSpec((BLK, F), lambda i: (i, 0)),
            pl.BlockSpec((NC, BLK, HF), lambda i: (0, i, 0)),
        ],
        out_shape=[
            jax.ShapeDtypeStruct((N, F), jnp.float32),
            jax.ShapeDtypeStruct((NC, N, HF), jnp.float32),
        ],
    )(agg2, agg2, dis, x1, out, W)


def kernel(x, edge_index, edge_weight, edge_attr, W0, b0, W1_0, W1_1):
    row = edge_index[0]
    col = edge_index[1]
    zeros_d = jnp.zeros((NPT8, DW), jnp.float32)
    zeros_e = jnp.zeros((NPT8, HF), jnp.float32)

    deg2 = _deg_kernel(col, edge_weight, zeros_d)
    x1, dis, outp2 = _tcb(x, W0, b0, deg2)

    out = x1
    for W in (W1_0, W1_1):
        agg2 = _edge_kernel(row, col, edge_weight,
                            outp2.reshape(NC * N, HF), zeros_e)
        out, outp2 = _tcc(agg2, dis, x1, out, W)
    return out


# speedup vs baseline: 20.7566x; 1.3705x over previous
import functools
import jax
import jax.numpy as jnp
from jax import lax
from jax.experimental import pallas as pl
from jax.experimental.pallas import tpu as pltpu
from jax.experimental.pallas import tpu_sc as plsc

N = 50000
E = 800000
H = 64
F = 64
HF = 32
DW = 16
ALPHA = 0.9

NC = 2
NS = 16
L = 16

CH = 400
EPT = E // NS
NCHUNK = EPT // CH

EPW = E // (NC * NS)
DEG_FULL = EPW // CH
DEG_REM_BASE = (NC * NS) * (DEG_FULL * CH)
NPT8 = 3128
NPTL = N - (NS - 1) * NPT8

BLK = 1000

_MESH = plsc.VectorSubcoreMesh(core_axis_name="c", subcore_axis_name="s")
_SC_PARAMS = pltpu.CompilerParams(use_tc_tiling_on_sc=False)


def _init_acc(zeros_hbm, acc_sh, s):
    off = s * NPT8

    @pl.when(s < NS - 1)
    def _():
        pltpu.sync_copy(zeros_hbm, acc_sh.at[pl.ds(off, NPT8)])

    @pl.when(s == NS - 1)
    def _():
        pltpu.sync_copy(zeros_hbm.at[pl.ds(0, NPTL)], acc_sh.at[pl.ds(off, NPTL)])


def _writeback_acc(acc_sh, out_hbm, c, s):
    off = s * NPT8

    @pl.when(s < NS - 1)
    def _():
        pltpu.sync_copy(acc_sh.at[pl.ds(off, NPT8)],
                        out_hbm.at[pl.ds(c * N + off, NPT8)])

    @pl.when(s == NS - 1)
    def _():
        pltpu.sync_copy(acc_sh.at[pl.ds(off, NPTL)],
                        out_hbm.at[pl.ds(c * N + off, NPTL)])


@functools.partial(
    pl.kernel,
    out_type=jax.ShapeDtypeStruct((NC * N, DW), jnp.float32),
    mesh=_MESH,
    compiler_params=_SC_PARAMS,
    scratch_types=[
        pltpu.VMEM((CH,), jnp.int32),
        pltpu.VMEM((CH,), jnp.float32),
        pltpu.VMEM((CH, DW), jnp.float32),
        pltpu.VMEM_SHARED((N, DW), jnp.float32),
    ],
)
def _deg_kernel(col_hbm, w_hbm, zeros_hbm, deg_hbm, col_v, w_v, rows_v, acc_sh):
    c = lax.axis_index("c")
    s = lax.axis_index("s")
    _init_acc(zeros_hbm, acc_sh, s)
    plsc.subcore_barrier()

    wid = c * NS + s

    def do_chunk(base):
        pltpu.sync_copy(col_hbm.at[pl.ds(base, CH)], col_v)
        pltpu.sync_copy(w_hbm.at[pl.ds(base, CH)], w_v)

        def splat(j, cc):
            wv16 = w_v[pl.ds(j * L, L)]
            for l in range(L):
                i = j * L + l
                rows_v[i, pl.ds(0, DW)] = jnp.full((DW,), wv16[l], jnp.float32)
            return cc

        lax.fori_loop(0, CH // L, splat, 0)
        pltpu.sync_copy(rows_v, acc_sh.at[col_v], add=True)

    def chunk(k, carry):
        do_chunk(wid * (DEG_FULL * CH) + k * CH)
        return carry

    lax.fori_loop(0, DEG_FULL, chunk, 0)

    @pl.when(wid < (E - DEG_REM_BASE) // CH)
    def _():
        do_chunk(DEG_REM_BASE + wid * CH)

    plsc.subcore_barrier()
    _writeback_acc(acc_sh, deg_hbm, c, s)


NI = 3
NR = 2


@functools.partial(
    pl.kernel,
    out_type=jax.ShapeDtypeStruct((NC * N, HF), jnp.float32),
    mesh=_MESH,
    compiler_params=_SC_PARAMS,
    scratch_types=(
        [pltpu.VMEM((CH,), jnp.int32) for _ in range(NI)]
        + [pltpu.VMEM((CH,), jnp.int32) for _ in range(NI)]
        + [pltpu.VMEM((CH,), jnp.float32) for _ in range(NI)]
        + [pltpu.VMEM((CH, HF), jnp.float32) for _ in range(NR)]
        + [pltpu.VMEM_SHARED((N, HF), jnp.float32)]
        + [pltpu.SemaphoreType.DMA for _ in range(2 * NI + 2 * NR)]
    ),
)
def _edge_kernel(row_hbm, col_hbm, w_hbm, outp2_hbm, zeros_hbm, agg_hbm,
                 *refs):
    gidx = refs[0:NI]
    colb = refs[NI:2 * NI]
    wb = refs[2 * NI:3 * NI]
    rows = refs[3 * NI:3 * NI + NR]
    acc_sh = refs[3 * NI + NR]
    p = 3 * NI + NR + 1
    sem_i = refs[p:p + NI]
    sem_g = refs[p + NI:p + NI + NR]
    sem_s = refs[p + NI + NR:p + NI + 2 * NR]

    c = lax.axis_index("c")
    s = lax.axis_index("s")
    _init_acc(zeros_hbm, acc_sh, s)
    plsc.subcore_barrier()

    cN = c * N

    def idx_load(k, bi):
        base = s * EPT + k * CH
        pltpu.async_copy(row_hbm.at[pl.ds(base, CH)], gidx[bi], sem_i[bi])
        pltpu.async_copy(col_hbm.at[pl.ds(base, CH)], colb[bi], sem_i[bi])
        pltpu.async_copy(w_hbm.at[pl.ds(base, CH)], wb[bi], sem_i[bi])

    def idx_wait(k, bi):
        base = s * EPT + k * CH
        pltpu.make_async_copy(row_hbm.at[pl.ds(base, CH)], gidx[bi], sem_i[bi]).wait()
        pltpu.make_async_copy(col_hbm.at[pl.ds(base, CH)], colb[bi], sem_i[bi]).wait()
        pltpu.make_async_copy(w_hbm.at[pl.ds(base, CH)], wb[bi], sem_i[bi]).wait()

    def fire_gather(bi, br):
        def addc(i, cc):
            gidx[bi][pl.ds(i * L, L)] = gidx[bi][pl.ds(i * L, L)] + cN
            return cc

        lax.fori_loop(0, CH // L, addc, 0, unroll=4)
        pltpu.async_copy(outp2_hbm.at[gidx[bi]], rows[br], sem_g[br])

    def wait_gather(bi, br):
        pltpu.make_async_copy(outp2_hbm.at[gidx[bi]], rows[br], sem_g[br]).wait()

    def fire_scatter(bi, br):
        pltpu.async_copy(rows[br], acc_sh.at[colb[bi]], sem_s[br], add=True)

    def wait_scatter(bi, br):
        pltpu.make_async_copy(rows[br], acc_sh.at[colb[bi]], sem_s[br]).wait()

    def scale(bi, br):
        def body(j, cc):
            wv16 = wb[bi][pl.ds(j * L, L)]
            for l in range(L):
                wv = wv16[l]
                i = j * L + l
                rows[br][i, pl.ds(0, L)] = rows[br][i, pl.ds(0, L)] * wv
                rows[br][i, pl.ds(L, L)] = rows[br][i, pl.ds(L, L)] * wv
            return cc

        lax.fori_loop(0, CH // L, body, 0)

    idx_load(0, 0)
    idx_load(1, 1)
    idx_wait(0, 0)
    fire_gather(0, 0)

    def group(g, carry):
        k0 = 2 * NI * g
        for j in range(2 * NI):
            k = k0 + j
            bi = j % NI
            br = j % NR
            bi1 = (j + 1) % NI
            br1 = (j + 1) % NR
            bi2 = (j + 2) % NI

            @pl.when(k + 1 < NCHUNK)
            def _():
                idx_wait(k + 1, bi1)

                @pl.when(k >= 1)
                def _():
                    wait_scatter((j - 1) % NI, br1)

                fire_gather(bi1, br1)

            wait_gather(bi, br)
            scale(bi, br)
            fire_scatter(bi, br)

            @pl.when(k + 2 < NCHUNK)
            def _():
                idx_load(k + 2, bi2)
        return carry

    lax.fori_loop(0, NCHUNK // (2 * NI), group, 0)
    for j in range(2 * NI * (NCHUNK // (2 * NI)), NCHUNK):
        k = j
        bi = j % NI
        br = j % NR
        bi1 = (j + 1) % NI
        br1 = (j + 1) % NR
        bi2 = (j + 2) % NI
        if k + 1 < NCHUNK:
            idx_wait(k + 1, bi1)
            wait_scatter((j - 1) % NI, br1)
            fire_gather(bi1, br1)
        wait_gather(bi, br)
        scale(bi, br)
        fire_scatter(bi, br)
        if k + 2 < NCHUNK:
            idx_load(k + 2, bi2)
    wait_scatter((NCHUNK - 2) % NI, (NCHUNK - 2) % NR)
    wait_scatter((NCHUNK - 1) % NI, (NCHUNK - 1) % NR)

    plsc.subcore_barrier()
    _writeback_acc(acc_sh, agg_hbm, c, s)


def _tcb_body(x_ref, w0_ref, b0_ref, degA_ref, degB_ref,
              x1_ref, dis_ref, outp2_ref):
    x1 = jax.nn.relu(
        jnp.dot(x_ref[...], w0_ref[...].T, preferred_element_type=jnp.float32)
        + b0_ref[...]
    )
    deg = degA_ref[...] + degB_ref[...]
    dis16 = jnp.where(deg > 0, lax.rsqrt(jnp.where(deg > 0, deg, 1.0)), 0.0)
    dis = jnp.concatenate([dis16, dis16], axis=1)
    x1_ref[...] = x1
    dis_ref[...] = dis
    outp2_ref[...] = jnp.stack([x1[:, :HF] * dis, x1[:, HF:] * dis], axis=0)


def _tcb(x, W0, b0, deg2):
    return pl.pallas_call(
        _tcb_body,
        grid=(N // BLK,),
        in_specs=[
            pl.BlockSpec((BLK, H), lambda i: (i, 0)),
            pl.BlockSpec((F, H), lambda i: (0, 0)),
            pl.BlockSpec((F,), lambda i: (0,)),
            pl.BlockSpec((BLK, DW), lambda i: (i, 0)),
            pl.BlockSpec((BLK, DW), lambda i: (N // BLK + i, 0)),
        ],
        out_specs=[
            pl.BlockSpec((BLK, F), lambda i: (i, 0)),
            pl.BlockSpec((BLK, HF), lambda i: (i, 0)),
            pl.BlockSpec((NC, BLK, HF), lambda i: (0, i, 0)),
        ],
        out_shape=[
            jax.ShapeDtypeStruct((N, F), jnp.float32),
            jax.ShapeDtypeStruct((N, HF), jnp.float32),
            jax.ShapeDtypeStruct((NC, N, HF), jnp.float32),
        ],
    )(x, W0, b0, deg2, deg2)


def _tcc_body(aggA_ref, aggB_ref, dis_ref, x1_ref, out_ref, w_ref,
              o_ref, outp2_ref):
    dis = dis_ref[...]
    agg = jnp.concatenate([aggA_ref[...] * dis, aggB_ref[...] * dis], axis=1)
    h = (1.0 - ALPHA) * agg + ALPHA * x1_ref[...]
    hw = jnp.dot(h, w_ref[...], preferred_element_type=jnp.float32)
    o = out_ref[...] + jax.nn.relu(hw)
    o_ref[...] = o
    outp2_ref[...] = jnp.stack([o[:, :HF] * dis, o[:, HF:] * dis], axis=0)


def _tcc(agg2, dis, x1, out, W):
    return pl.pallas_call(
        _tcc_body,
        grid=(N // BLK,),
        in_specs=[
            pl.BlockSpec((BLK, HF), lambda i: (i, 0)),
            pl.BlockSpec((BLK, HF), lambda i: (N // BLK + i, 0)),
            pl.BlockSpec((BLK, HF), lambda i: (i, 0)),
            pl.BlockSpec((BLK, F), lambda i: (i, 0)),
            pl.BlockSpec((BLK, F), lambda i: (i, 0)),
            pl.BlockSpec((F, F), lambda i: (0, 0)),
        ],
        out_specs=[
            pl.BlockSpec((BLK, F), lambda i: (i, 0)),
            pl.BlockSpec((NC, BLK, HF), lambda i: (0, i, 0)),
        ],
        out_shape=[
            jax.ShapeDtypeStruct((N, F), jnp.float32),
            jax.ShapeDtypeStruct((NC, N, HF), jnp.float32),
        ],
    )(agg2, agg2, dis, x1, out, W)


def kernel(x, edge_index, edge_weight, edge_attr, W0, b0, W1_0, W1_1):
    row = edge_index[0]
    col = edge_index[1]
    zeros_d = jnp.zeros((NPT8, DW), jnp.float32)
    zeros_e = jnp.zeros((NPT8, HF), jnp.float32)

    deg2 = _deg_kernel(col, edge_weight, zeros_d)
    x1, dis, outp2 = _tcb(x, W0, b0, deg2)

    out = x1
    for W in (W1_0, W1_1):
        agg2 = _edge_kernel(row, col, edge_weight,
                            outp2.reshape(NC * N, HF), zeros_e)
        out, outp2 = _tcc(agg2, dis, x1, out, W)
    return out


# speedup vs baseline: 22.7548x; 1.0963x over previous
import functools
import jax
import jax.numpy as jnp
from jax import lax
from jax.experimental import pallas as pl
from jax.experimental.pallas import tpu as pltpu
from jax.experimental.pallas import tpu_sc as plsc

N = 50000
E = 800000
H = 64
F = 64
HF = 32
DW = 16
ALPHA = 0.9

NC = 2
NS = 16
L = 16

CH = 400
EPT = E // NS
NCHUNK = EPT // CH

EPW = E // (NC * NS)
DEG_FULL = EPW // CH
DEG_REM_BASE = (NC * NS) * (DEG_FULL * CH)
NPT8 = 3128
NPTL = N - (NS - 1) * NPT8

BLK = 1000

_MESH = plsc.VectorSubcoreMesh(core_axis_name="c", subcore_axis_name="s")
_SC_PARAMS = pltpu.CompilerParams(use_tc_tiling_on_sc=False)


def _init_acc(zeros_hbm, acc_sh, s):
    off = s * NPT8

    @pl.when(s < NS - 1)
    def _():
        pltpu.sync_copy(zeros_hbm, acc_sh.at[pl.ds(off, NPT8)])

    @pl.when(s == NS - 1)
    def _():
        pltpu.sync_copy(zeros_hbm.at[pl.ds(0, NPTL)], acc_sh.at[pl.ds(off, NPTL)])


def _writeback_acc(acc_sh, out_hbm, c, s):
    off = s * NPT8

    @pl.when(s < NS - 1)
    def _():
        pltpu.sync_copy(acc_sh.at[pl.ds(off, NPT8)],
                        out_hbm.at[pl.ds(c * N + off, NPT8)])

    @pl.when(s == NS - 1)
    def _():
        pltpu.sync_copy(acc_sh.at[pl.ds(off, NPTL)],
                        out_hbm.at[pl.ds(c * N + off, NPTL)])


DNI = 3
DNR = 2


@functools.partial(
    pl.kernel,
    out_type=jax.ShapeDtypeStruct((NC * N, DW), jnp.float32),
    mesh=_MESH,
    compiler_params=_SC_PARAMS,
    scratch_types=(
        [pltpu.VMEM((CH,), jnp.int32) for _ in range(DNI)]
        + [pltpu.VMEM((CH,), jnp.float32) for _ in range(DNI)]
        + [pltpu.VMEM((CH, DW), jnp.float32) for _ in range(DNR)]
        + [pltpu.VMEM_SHARED((N, DW), jnp.float32)]
        + [pltpu.SemaphoreType.DMA for _ in range(DNI + DNR)]
    ),
)
def _deg_kernel(col_hbm, w_hbm, zeros_hbm, deg_hbm, *refs):
    colb = refs[0:DNI]
    wb = refs[DNI:2 * DNI]
    rows = refs[2 * DNI:2 * DNI + DNR]
    acc_sh = refs[2 * DNI + DNR]
    p = 2 * DNI + DNR + 1
    sem_i = refs[p:p + DNI]
    sem_s = refs[p + DNI:p + DNI + DNR]

    c = lax.axis_index("c")
    s = lax.axis_index("s")
    _init_acc(zeros_hbm, acc_sh, s)
    plsc.subcore_barrier()

    wid = c * NS + s
    base0 = wid * (DEG_FULL * CH)

    def idx_load(k, bi):
        base = base0 + k * CH
        pltpu.async_copy(col_hbm.at[pl.ds(base, CH)], colb[bi], sem_i[bi])
        pltpu.async_copy(w_hbm.at[pl.ds(base, CH)], wb[bi], sem_i[bi])

    def idx_wait(k, bi):
        base = base0 + k * CH
        pltpu.make_async_copy(col_hbm.at[pl.ds(base, CH)], colb[bi], sem_i[bi]).wait()
        pltpu.make_async_copy(w_hbm.at[pl.ds(base, CH)], wb[bi], sem_i[bi]).wait()

    def splat(bi, br):
        @plsc.parallel_loop(0, CH // L, unroll=2)
        def body(j):
            wv16 = wb[bi][pl.ds(j * L, L)]
            for l in range(L):
                i = j * L + l
                rows[br][i, pl.ds(0, DW)] = jnp.full((DW,), wv16[l], jnp.float32)

    def fire_scatter(bi, br):
        pltpu.async_copy(rows[br], acc_sh.at[colb[bi]], sem_s[br], add=True)

    def wait_scatter(bi, br):
        pltpu.make_async_copy(rows[br], acc_sh.at[colb[bi]], sem_s[br]).wait()

    idx_load(0, 0)
    idx_load(1, 1)

    def group(g, carry):
        k0 = 2 * DNI * g
        for j in range(2 * DNI):
            k = k0 + j
            bi = j % DNI
            br = j % DNR
            bi1 = (j + 1) % DNI
            idx_wait(k, bi)
            splat(bi, br)
            fire_scatter(bi, br)

            @pl.when(k + 2 < DEG_FULL)
            def _():
                @pl.when(k >= 1)
                def _():
                    wait_scatter((j - 1) % DNI, (j - 1) % DNR)

                idx_load(k + 2, (j + 2) % DNI)
        return carry

    lax.fori_loop(0, DEG_FULL // (2 * DNI), group, 0)
    idx_wait(60, 60 % DNI)
    splat(60 % DNI, 0)
    fire_scatter(60 % DNI, 0)
    idx_wait(61, 61 % DNI)
    wait_scatter(59 % DNI, 1)
    splat(61 % DNI, 1)
    fire_scatter(61 % DNI, 1)
    wait_scatter(60 % DNI, 0)
    wait_scatter(61 % DNI, 1)

    @pl.when(wid < (E - DEG_REM_BASE) // CH)
    def _():
        base = DEG_REM_BASE + wid * CH
        pltpu.async_copy(col_hbm.at[pl.ds(base, CH)], colb[0], sem_i[0])
        pltpu.async_copy(w_hbm.at[pl.ds(base, CH)], wb[0], sem_i[0])
        pltpu.make_async_copy(col_hbm.at[pl.ds(base, CH)], colb[0], sem_i[0]).wait()
        pltpu.make_async_copy(w_hbm.at[pl.ds(base, CH)], wb[0], sem_i[0]).wait()
        splat(0, 0)
        pltpu.sync_copy(rows[0], acc_sh.at[colb[0]], add=True)

    plsc.subcore_barrier()
    _writeback_acc(acc_sh, deg_hbm, c, s)


NI = 3
NR = 2


@functools.partial(
    pl.kernel,
    out_type=jax.ShapeDtypeStruct((NC * N, HF), jnp.float32),
    mesh=_MESH,
    compiler_params=_SC_PARAMS,
    scratch_types=(
        [pltpu.VMEM((CH,), jnp.int32) for _ in range(NI)]
        + [pltpu.VMEM((CH,), jnp.int32) for _ in range(NI)]
        + [pltpu.VMEM((CH,), jnp.float32) for _ in range(NI)]
        + [pltpu.VMEM((CH, HF), jnp.float32) for _ in range(NR)]
        + [pltpu.VMEM_SHARED((N, HF), jnp.float32)]
        + [pltpu.SemaphoreType.DMA for _ in range(2 * NI + 2 * NR)]
    ),
)
def _edge_kernel(row_hbm, col_hbm, w_hbm, outp2_hbm, zeros_hbm, agg_hbm,
                 *refs):
    gidx = refs[0:NI]
    colb = refs[NI:2 * NI]
    wb = refs[2 * NI:3 * NI]
    rows = refs[3 * NI:3 * NI + NR]
    acc_sh = refs[3 * NI + NR]
    p = 3 * NI + NR + 1
    sem_i = refs[p:p + NI]
    sem_g = refs[p + NI:p + NI + NR]
    sem_s = refs[p + NI + NR:p + NI + 2 * NR]

    c = lax.axis_index("c")
    s = lax.axis_index("s")
    _init_acc(zeros_hbm, acc_sh, s)
    plsc.subcore_barrier()

    cN = c * N

    def idx_load(k, bi):
        base = s * EPT + k * CH
        pltpu.async_copy(row_hbm.at[pl.ds(base, CH)], gidx[bi], sem_i[bi])
        pltpu.async_copy(col_hbm.at[pl.ds(base, CH)], colb[bi], sem_i[bi])
        pltpu.async_copy(w_hbm.at[pl.ds(base, CH)], wb[bi], sem_i[bi])

    def idx_wait(k, bi):
        base = s * EPT + k * CH
        pltpu.make_async_copy(row_hbm.at[pl.ds(base, CH)], gidx[bi], sem_i[bi]).wait()
        pltpu.make_async_copy(col_hbm.at[pl.ds(base, CH)], colb[bi], sem_i[bi]).wait()
        pltpu.make_async_copy(w_hbm.at[pl.ds(base, CH)], wb[bi], sem_i[bi]).wait()

    def fire_gather(bi, br):
        def addc(i, cc):
            gidx[bi][pl.ds(i * L, L)] = gidx[bi][pl.ds(i * L, L)] + cN
            return cc

        lax.fori_loop(0, CH // L, addc, 0, unroll=4)
        pltpu.async_copy(outp2_hbm.at[gidx[bi]], rows[br], sem_g[br])

    def wait_gather(bi, br):
        pltpu.make_async_copy(outp2_hbm.at[gidx[bi]], rows[br], sem_g[br]).wait()

    def fire_scatter(bi, br):
        pltpu.async_copy(rows[br], acc_sh.at[colb[bi]], sem_s[br], add=True)

    def wait_scatter(bi, br):
        pltpu.make_async_copy(rows[br], acc_sh.at[colb[bi]], sem_s[br]).wait()

    def scale(bi, br):
        @plsc.parallel_loop(0, CH // L, unroll=2)
        def body(j):
            wv16 = wb[bi][pl.ds(j * L, L)]
            for l in range(L):
                wv = wv16[l]
                i = j * L + l
                rows[br][i, pl.ds(0, L)] = rows[br][i, pl.ds(0, L)] * wv
                rows[br][i, pl.ds(L, L)] = rows[br][i, pl.ds(L, L)] * wv

    idx_load(0, 0)
    idx_load(1, 1)
    idx_wait(0, 0)
    fire_gather(0, 0)

    def group(g, carry):
        k0 = 2 * NI * g
        for j in range(2 * NI):
            k = k0 + j
            bi = j % NI
            br = j % NR
            bi1 = (j + 1) % NI
            br1 = (j + 1) % NR
            bi2 = (j + 2) % NI

            @pl.when(k + 1 < NCHUNK)
            def _():
                idx_wait(k + 1, bi1)

                @pl.when(k >= 1)
                def _():
                    wait_scatter((j - 1) % NI, br1)

                fire_gather(bi1, br1)

            wait_gather(bi, br)
            scale(bi, br)
            fire_scatter(bi, br)

            @pl.when(k + 2 < NCHUNK)
            def _():
                idx_load(k + 2, bi2)
        return carry

    lax.fori_loop(0, NCHUNK // (2 * NI), group, 0)
    for j in range(2 * NI * (NCHUNK // (2 * NI)), NCHUNK):
        k = j
        bi = j % NI
        br = j % NR
        bi1 = (j + 1) % NI
        br1 = (j + 1) % NR
        bi2 = (j + 2) % NI
        if k + 1 < NCHUNK:
            idx_wait(k + 1, bi1)
            wait_scatter((j - 1) % NI, br1)
            fire_gather(bi1, br1)
        wait_gather(bi, br)
        scale(bi, br)
        fire_scatter(bi, br)
        if k + 2 < NCHUNK:
            idx_load(k + 2, bi2)
    wait_scatter((NCHUNK - 2) % NI, (NCHUNK - 2) % NR)
    wait_scatter((NCHUNK - 1) % NI, (NCHUNK - 1) % NR)

    plsc.subcore_barrier()
    _writeback_acc(acc_sh, agg_hbm, c, s)


def _tcb_body(x_ref, w0_ref, b0_ref, degA_ref, degB_ref,
              x1_ref, dis_ref, outp2_ref):
    x1 = jax.nn.relu(
        jnp.dot(x_ref[...], w0_ref[...].T, preferred_element_type=jnp.float32)
        + b0_ref[...]
    )
    deg = degA_ref[...] + degB_ref[...]
    dis16 = jnp.where(deg > 0, lax.rsqrt(jnp.where(deg > 0, deg, 1.0)), 0.0)
    dis = jnp.concatenate([dis16, dis16], axis=1)
    x1_ref[...] = x1
    dis_ref[...] = dis
    outp2_ref[...] = jnp.stack([x1[:, :HF] * dis, x1[:, HF:] * dis], axis=0)


def _tcb(x, W0, b0, deg2):
    return pl.pallas_call(
        _tcb_body,
        grid=(N // BLK,),
        in_specs=[
            pl.BlockSpec((BLK, H), lambda i: (i, 0)),
            pl.BlockSpec((F, H), lambda i: (0, 0)),
            pl.BlockSpec((F,), lambda i: (0,)),
            pl.BlockSpec((BLK, DW), lambda i: (i, 0)),
            pl.BlockSpec((BLK, DW), lambda i: (N // BLK + i, 0)),
        ],
        out_specs=[
            pl.BlockSpec((BLK, F), lambda i: (i, 0)),
            pl.BlockSpec((BLK, HF), lambda i: (i, 0)),
            pl.BlockSpec((NC, BLK, HF), lambda i: (0, i, 0)),
        ],
        out_shape=[
            jax.ShapeDtypeStruct((N, F), jnp.float32),
            jax.ShapeDtypeStruct((N, HF), jnp.float32),
            jax.ShapeDtypeStruct((NC, N, HF), jnp.float32),
        ],
    )(x, W0, b0, deg2, deg2)


def _tcc_body(aggA_ref, aggB_ref, dis_ref, x1_ref, out_ref, w_ref,
              o_ref, outp2_ref):
    dis = dis_ref[...]
    agg = jnp.concatenate([aggA_ref[...] * dis, aggB_ref[...] * dis], axis=1)
    h = (1.0 - ALPHA) * agg + ALPHA * x1_ref[...]
    hw = jnp.dot(h, w_ref[...], preferred_element_type=jnp.float32)
    o = out_ref[...] + jax.nn.relu(hw)
    o_ref[...] = o
    outp2_ref[...] = jnp.stack([o[:, :HF] * dis, o[:, HF:] * dis], axis=0)


def _tcc(agg2, dis, x1, out, W):
    return pl.pallas_call(
        _tcc_body,
        grid=(N // BLK,),
        in_specs=[
            pl.BlockSpec((BLK, HF), lambda i: (i, 0)),
            pl.BlockSpec((BLK, HF), lambda i: (N // BLK + i, 0)),
            pl.BlockSpec((BLK, HF), lambda i: (i, 0)),
            pl.BlockSpec((BLK, F), lambda i: (i, 0)),
            pl.BlockSpec((BLK, F), lambda i: (i, 0)),
            pl.BlockSpec((F, F), lambda i: (0, 0)),
        ],
        out_specs=[
            pl.BlockSpec((BLK, F), lambda i: (i, 0)),
            pl.BlockSpec((NC, BLK, HF), lambda i: (0, i, 0)),
        ],
        out_shape=[
            jax.ShapeDtypeStruct((N, F), jnp.float32),
            jax.ShapeDtypeStruct((NC, N, HF), jnp.float32),
        ],
    )(agg2, agg2, dis, x1, out, W)


def kernel(x, edge_index, edge_weight, edge_attr, W0, b0, W1_0, W1_1):
    row = edge_index[0]
    col = edge_index[1]
    zeros_d = jnp.zeros((NPT8, DW), jnp.float32)
    zeros_e = jnp.zeros((NPT8, HF), jnp.float32)

    deg2 = _deg_kernel(col, edge_weight, zeros_d)
    x1, dis, outp2 = _tcb(x, W0, b0, deg2)

    out = x1
    for W in (W1_0, W1_1):
        agg2 = _edge_kernel(row, col, edge_weight,
                            outp2.reshape(NC * N, HF), zeros_e)
        out, outp2 = _tcc(agg2, dis, x1, out, W)
    return out
